# baseline (speedup 1.0000x reference)
import jax
import jax.numpy as jnp
from jax import lax
from jax.experimental import pallas as pl
from jax.experimental.pallas import tpu as pltpu
from jax.experimental.pallas import tpu_sc as plsc

_N = 10000
_E = 320000
_D = 128
_NC = 2
_NS = 16
_NW = _NC * _NS
_CH = 64
_NCH = 160
_PC = 16
_NPH = _NCH // _PC
_EPW = _NCH * _CH
_EPAD = _NW * _EPW - _E
_NP = 10240
_RPT = _NP // _NS


def _segsum_body(y_hbm, src_hbm, dst_hbm, out_hbm,
                 src_v, dst_v, rows0, acc_sh, gsem0):
    c = lax.axis_index("c")
    s = lax.axis_index("s")
    w = s * _NC + c
    zrow = jnp.zeros((16,), jnp.float32)

    def _zfill(r, carry):
        for j in range(_D // 16):
            rows0[r, pl.ds(j * 16, 16)] = zrow
        return carry

    lax.fori_loop(0, _CH, _zfill, 0)
    base = s * _RPT

    def _zero(i, carry):
        pltpu.sync_copy(rows0, acc_sh.at[pl.ds(base + i * _CH, _CH)])
        return carry

    lax.fori_loop(0, _RPT // _CH, _zero, 0)
    plsc.subcore_barrier()

    pltpu.sync_copy(src_hbm.at[w], src_v)
    pltpu.sync_copy(dst_hbm.at[w], dst_v)

    def _edge(j, carry):
        pltpu.async_copy(y_hbm.at[src_v.at[j]], rows0, gsem0).wait()
        pltpu.sync_copy(rows0, acc_sh.at[dst_v.at[j]], add=True)
        return carry

    lax.fori_loop(0, _NCH, _edge, 0)
    plsc.subcore_barrier()
    pltpu.sync_copy(acc_sh.at[pl.ds(base, _RPT)], out_hbm.at[c, pl.ds(base, _RPT)])


def _segsum(y, src3, dst3):
    mesh = plsc.VectorSubcoreMesh(core_axis_name="c", subcore_axis_name="s")
    f = pl.kernel(
        _segsum_body,
        mesh=mesh,
        out_type=jax.ShapeDtypeStruct((_NC, _NP, _D), jnp.float32),
        scratch_types=[
            pltpu.VMEM((_NCH, _CH), jnp.int32),
            pltpu.VMEM((_NCH, _CH), jnp.int32),
            pltpu.VMEM((_CH, _D), jnp.float32),
            pltpu.VMEM_SHARED((_NP, _D), jnp.float32),
            pltpu.SemaphoreType.DMA,
        ],
    )
    return f(y, src3, dst3)


_BR = 2000


def _mlp_body(p_ref, w0_ref, b0_ref, w1_ref, b1_ref, o_ref):
    agg = p_ref[0] + p_ref[1]
    h = jnp.dot(agg, w0_ref[...], preferred_element_type=jnp.float32) + b0_ref[...]
    h = jnp.maximum(h, 0.0)
    o_ref[...] = jnp.dot(h, w1_ref[...], preferred_element_type=jnp.float32) + b1_ref[...]


def _mlp(p, w0, b0, w1, b1):
    grid = (_N // _BR,)
    return pl.pallas_call(
        _mlp_body,
        grid=grid,
        in_specs=[
            pl.BlockSpec((_NC, _BR, _D), lambda i: (0, i, 0)),
            pl.BlockSpec((_D, _D), lambda i: (0, 0)),
            pl.BlockSpec((1, _D), lambda i: (0, 0)),
            pl.BlockSpec((_D, _D), lambda i: (0, 0)),
            pl.BlockSpec((1, _D), lambda i: (0, 0)),
        ],
        out_specs=pl.BlockSpec((_BR, _D), lambda i: (i, 0)),
        out_shape=jax.ShapeDtypeStruct((_N, _D), jnp.float32),
    )(p, w0, b0, w1, b1)


def _final_body(p_ref, x_ref, wf_ref, w0_ref, b0_ref, w1_ref, b1_ref,
                a_ref, bm_ref, c_ref, mb0_ref, wo_ref, mbo_ref, o_ref):
    agg = p_ref[0] + p_ref[1]
    h = jnp.dot(agg, w0_ref[...], preferred_element_type=jnp.float32) + b0_ref[...]
    h = jnp.maximum(h, 0.0)
    y2 = jnp.dot(h, w1_ref[...], preferred_element_type=jnp.float32) + b1_ref[...]
    t = (jnp.dot(y2, a_ref[...], preferred_element_type=jnp.float32)
         + jnp.dot(x_ref[...], bm_ref[...], preferred_element_type=jnp.float32)
         + jnp.dot(wf_ref[...], c_ref[...], preferred_element_type=jnp.float32)
         + mb0_ref[...])
    t = jnp.maximum(t, 0.0)
    o_ref[...] = jnp.dot(t, wo_ref[...], preferred_element_type=jnp.float32) + mbo_ref[...]


def _final(p, x, wf, w0, b0, w1, b1, a, bm, cm, mb0, wo, mbo):
    grid = (_N // _BR,)
    mat = pl.BlockSpec((_D, _D), lambda i: (0, 0))
    vec = pl.BlockSpec((1, _D), lambda i: (0, 0))
    row = pl.BlockSpec((_BR, _D), lambda i: (i, 0))
    return pl.pallas_call(
        _final_body,
        grid=grid,
        in_specs=[
            pl.BlockSpec((_NC, _BR, _D), lambda i: (0, i, 0)),
            row, row, mat, vec, mat, vec, mat, mat, mat, vec, mat, vec,
        ],
        out_specs=row,
        out_shape=jax.ShapeDtypeStruct((_N, _D), jnp.float32),
    )(p, x, wf, w0, b0, w1, b1, a, bm, cm, mb0, wo, mbo)


def kernel(node_attributes, edge_index, X, W,
           g0W0, g0b0, g0W1, g0b1, g1W0, g1b0, g1W1, g1b1,
           mW0, mb0, mWo, mbo):
    pad_dst = _N + (jnp.arange(_EPAD, dtype=jnp.int32) % (_NP - _N))
    src3 = jnp.concatenate([edge_index[0], jnp.zeros((_EPAD,), jnp.int32)]
                           ).reshape(_NW, _NCH, _CH)
    dst3 = jnp.concatenate([edge_index[1], pad_dst]).reshape(_NW, _NCH, _CH)

    p1 = _segsum(node_attributes, src3, dst3)[:, :_N]
    y1 = _mlp(p1, g0W0, g0b0.reshape(1, _D), g0W1, g0b1.reshape(1, _D))
    p2 = _segsum(y1, src3, dst3)[:, :_N]

    a = mW0[:_D]
    bm = mW0[_D:2 * _D]
    cm = jnp.zeros((_D, _D), jnp.float32).at[:7].set(mW0[2 * _D:])
    wf = jnp.pad(W, ((0, 0), (0, _D - 7)))
    wo = jnp.zeros((_D, _D), jnp.float32).at[:, :4].set(mWo)
    mbo_p = jnp.zeros((1, _D), jnp.float32).at[0, :4].set(mbo)
    out = _final(p2, X, wf, g1W0, g1b0.reshape(1, _D), g1W1, g1b1.reshape(1, _D),
                 a, bm, cm, mb0.reshape(1, _D), wo, mbo_p)
    return out[:, :4]


# speedup vs baseline: 2.6158x; 2.6158x over previous
import jax
import jax.numpy as jnp
from jax import lax
from jax.experimental import pallas as pl
from jax.experimental.pallas import tpu as pltpu
from jax.experimental.pallas import tpu_sc as plsc

_N = 10000
_E = 320000
_D = 128
_NC = 2
_NS = 16
_NW = _NC * _NS
_CH = 80
_NCH = 125
_PC = 16
_NPH = _NCH // _PC
_EPW = _NCH * _CH
_EPAD = _NW * _EPW - _E
_NP = 10240
_RPT = _NP // _NS


def _segsum_body(y_hbm, src_hbm, dst_hbm, out_hbm,
                 src_v, dst_v, rows0, acc_sh, gsem0):
    c = lax.axis_index("c")
    s = lax.axis_index("s")
    w = s * _NC + c
    zrow = jnp.zeros((16,), jnp.float32)

    def _zfill(r, carry):
        for j in range(_D // 16):
            rows0[r, pl.ds(j * 16, 16)] = zrow
        return carry

    lax.fori_loop(0, _CH, _zfill, 0)
    base = s * _RPT

    def _zero(i, carry):
        pltpu.sync_copy(rows0, acc_sh.at[pl.ds(base + i * _CH, _CH)])
        return carry

    lax.fori_loop(0, _RPT // _CH, _zero, 0)
    plsc.subcore_barrier()

    pltpu.sync_copy(src_hbm.at[w], src_v)
    pltpu.sync_copy(dst_hbm.at[w], dst_v)

    def _edge(j, carry):
        pltpu.async_copy(y_hbm.at[src_v.at[j]], rows0, gsem0).wait()
        pltpu.sync_copy(rows0, acc_sh.at[dst_v.at[j]], add=True)
        return carry

    lax.fori_loop(0, _NCH, _edge, 0)
    plsc.subcore_barrier()
    pltpu.sync_copy(acc_sh.at[pl.ds(base, _RPT)], out_hbm.at[c, pl.ds(base, _RPT)])


def _segsum(y, src3, dst3):
    mesh = plsc.VectorSubcoreMesh(core_axis_name="c", subcore_axis_name="s")
    f = pl.kernel(
        _segsum_body,
        mesh=mesh,
        out_type=jax.ShapeDtypeStruct((_NC, _NP, _D), jnp.float32),
        scratch_types=[
            pltpu.VMEM((_NCH, _CH), jnp.int32),
            pltpu.VMEM((_NCH, _CH), jnp.int32),
            pltpu.VMEM((_CH, _D), jnp.float32),
            pltpu.VMEM_SHARED((_NP, _D), jnp.float32),
            pltpu.SemaphoreType.DMA,
        ],
    )
    return f(y, src3, dst3)


_BR = 2000


def _mlp_body(p_ref, w0_ref, b0_ref, w1_ref, b1_ref, o_ref):
    agg = p_ref[0] + p_ref[1]
    h = jnp.dot(agg, w0_ref[...], preferred_element_type=jnp.float32) + b0_ref[...]
    h = jnp.maximum(h, 0.0)
    o_ref[...] = jnp.dot(h, w1_ref[...], preferred_element_type=jnp.float32) + b1_ref[...]


def _mlp(p, w0, b0, w1, b1):
    grid = (_N // _BR,)
    return pl.pallas_call(
        _mlp_body,
        grid=grid,
        in_specs=[
            pl.BlockSpec((_NC, _BR, _D), lambda i: (0, i, 0)),
            pl.BlockSpec((_D, _D), lambda i: (0, 0)),
            pl.BlockSpec((1, _D), lambda i: (0, 0)),
            pl.BlockSpec((_D, _D), lambda i: (0, 0)),
            pl.BlockSpec((1, _D), lambda i: (0, 0)),
        ],
        out_specs=pl.BlockSpec((_BR, _D), lambda i: (i, 0)),
        out_shape=jax.ShapeDtypeStruct((_N, _D), jnp.float32),
    )(p, w0, b0, w1, b1)


def _final_body(p_ref, x_ref, wf_ref, w0_ref, b0_ref, w1_ref, b1_ref,
                a_ref, bm_ref, c_ref, mb0_ref, wo_ref, mbo_ref, o_ref):
    agg = p_ref[0] + p_ref[1]
    h = jnp.dot(agg, w0_ref[...], preferred_element_type=jnp.float32) + b0_ref[...]
    h = jnp.maximum(h, 0.0)
    y2 = jnp.dot(h, w1_ref[...], preferred_element_type=jnp.float32) + b1_ref[...]
    t = (jnp.dot(y2, a_ref[...], preferred_element_type=jnp.float32)
         + jnp.dot(x_ref[...], bm_ref[...], preferred_element_type=jnp.float32)
         + jnp.dot(wf_ref[...], c_ref[...], preferred_element_type=jnp.float32)
         + mb0_ref[...])
    t = jnp.maximum(t, 0.0)
    o_ref[...] = jnp.dot(t, wo_ref[...], preferred_element_type=jnp.float32) + mbo_ref[...]


def _final(p, x, wf, w0, b0, w1, b1, a, bm, cm, mb0, wo, mbo):
    grid = (_N // _BR,)
    mat = pl.BlockSpec((_D, _D), lambda i: (0, 0))
    vec = pl.BlockSpec((1, _D), lambda i: (0, 0))
    row = pl.BlockSpec((_BR, _D), lambda i: (i, 0))
    return pl.pallas_call(
        _final_body,
        grid=grid,
        in_specs=[
            pl.BlockSpec((_NC, _BR, _D), lambda i: (0, i, 0)),
            row, row, mat, vec, mat, vec, mat, mat, mat, vec, mat, vec,
        ],
        out_specs=row,
        out_shape=jax.ShapeDtypeStruct((_N, _D), jnp.float32),
    )(p, x, wf, w0, b0, w1, b1, a, bm, cm, mb0, wo, mbo)


def kernel(node_attributes, edge_index, X, W,
           g0W0, g0b0, g0W1, g0b1, g1W0, g1b0, g1W1, g1b1,
           mW0, mb0, mWo, mbo):
    if _EPAD:
        pad_dst = _N + (jnp.arange(_EPAD, dtype=jnp.int32) % (_NP - _N))
        src_p = jnp.concatenate([edge_index[0], jnp.zeros((_EPAD,), jnp.int32)])
        dst_p = jnp.concatenate([edge_index[1], pad_dst])
    else:
        src_p, dst_p = edge_index[0], edge_index[1]
    src3 = src_p.reshape(_NW, _NCH, _CH)
    dst3 = dst_p.reshape(_NW, _NCH, _CH)

    p1 = _segsum(node_attributes, src3, dst3)[:, :_N]
    y1 = _mlp(p1, g0W0, g0b0.reshape(1, _D), g0W1, g0b1.reshape(1, _D))
    p2 = _segsum(y1, src3, dst3)[:, :_N]

    a = mW0[:_D]
    bm = mW0[_D:2 * _D]
    cm = jnp.zeros((_D, _D), jnp.float32).at[:7].set(mW0[2 * _D:])
    wf = jnp.pad(W, ((0, 0), (0, _D - 7)))
    wo = jnp.zeros((_D, _D), jnp.float32).at[:, :4].set(mWo)
    mbo_p = jnp.zeros((1, _D), jnp.float32).at[0, :4].set(mbo)
    out = _final(p2, X, wf, g1W0, g1b0.reshape(1, _D), g1W1, g1b1.reshape(1, _D),
                 a, bm, cm, mb0.reshape(1, _D), wo, mbo_p)
    return out[:, :4]


# speedup vs baseline: 2.9960x; 1.1453x over previous
import jax
import jax.numpy as jnp
from jax import lax
from jax.experimental import pallas as pl
from jax.experimental.pallas import tpu as pltpu
from jax.experimental.pallas import tpu_sc as plsc

_N = 10000
_E = 320000
_D = 128
_NC = 2
_NS = 16
_NW = _NC * _NS
_CH = 128
_NCH = 80
_PC = 16
_NPH = _NCH // _PC
_EPW = _NCH * _CH
_EPAD = _NW * _EPW - _E
_NP = 10240
_RPT = _NP // _NS


def _segsum_body(y_hbm, src_hbm, dst_hbm, out_hbm,
                 src_v, dst_v, rows0, acc_sh, gsem0):
    c = lax.axis_index("c")
    s = lax.axis_index("s")
    w = s * _NC + c
    zrow = jnp.zeros((16,), jnp.float32)

    def _zfill(r, carry):
        for j in range(_D // 16):
            rows0[r, pl.ds(j * 16, 16)] = zrow
        return carry

    lax.fori_loop(0, _CH, _zfill, 0)
    base = s * _RPT

    def _zero(i, carry):
        pltpu.sync_copy(rows0, acc_sh.at[pl.ds(base + i * _CH, _CH)])
        return carry

    lax.fori_loop(0, _RPT // _CH, _zero, 0)
    plsc.subcore_barrier()

    pltpu.sync_copy(src_hbm.at[w], src_v)
    pltpu.sync_copy(dst_hbm.at[w], dst_v)

    def _edge(j, carry):
        pltpu.async_copy(y_hbm.at[src_v.at[j]], rows0, gsem0).wait()
        pltpu.sync_copy(rows0, acc_sh.at[dst_v.at[j]], add=True)
        return carry

    lax.fori_loop(0, _NCH, _edge, 0)
    plsc.subcore_barrier()
    pltpu.sync_copy(acc_sh.at[pl.ds(base, _RPT)], out_hbm.at[c, pl.ds(base, _RPT)])


def _segsum(y, src3, dst3):
    mesh = plsc.VectorSubcoreMesh(core_axis_name="c", subcore_axis_name="s")
    f = pl.kernel(
        _segsum_body,
        mesh=mesh,
        out_type=jax.ShapeDtypeStruct((_NC, _NP, _D), jnp.float32),
        scratch_types=[
            pltpu.VMEM((_NCH, _CH), jnp.int32),
            pltpu.VMEM((_NCH, _CH), jnp.int32),
            pltpu.VMEM((_CH, _D), jnp.float32),
            pltpu.VMEM_SHARED((_NP, _D), jnp.float32),
            pltpu.SemaphoreType.DMA,
        ],
    )
    return f(y, src3, dst3)


_BR = 2000


def _mlp_body(p_ref, w0_ref, b0_ref, w1_ref, b1_ref, o_ref):
    agg = p_ref[0] + p_ref[1]
    h = jnp.dot(agg, w0_ref[...], preferred_element_type=jnp.float32) + b0_ref[...]
    h = jnp.maximum(h, 0.0)
    o_ref[...] = jnp.dot(h, w1_ref[...], preferred_element_type=jnp.float32) + b1_ref[...]


def _mlp(p, w0, b0, w1, b1):
    grid = (_N // _BR,)
    return pl.pallas_call(
        _mlp_body,
        grid=grid,
        in_specs=[
            pl.BlockSpec((_NC, _BR, _D), lambda i: (0, i, 0)),
            pl.BlockSpec((_D, _D), lambda i: (0, 0)),
            pl.BlockSpec((1, _D), lambda i: (0, 0)),
            pl.BlockSpec((_D, _D), lambda i: (0, 0)),
            pl.BlockSpec((1, _D), lambda i: (0, 0)),
        ],
        out_specs=pl.BlockSpec((_BR, _D), lambda i: (i, 0)),
        out_shape=jax.ShapeDtypeStruct((_N, _D), jnp.float32),
    )(p, w0, b0, w1, b1)


def _final_body(p_ref, x_ref, wf_ref, w0_ref, b0_ref, w1_ref, b1_ref,
                a_ref, bm_ref, c_ref, mb0_ref, wo_ref, mbo_ref, o_ref):
    agg = p_ref[0] + p_ref[1]
    h = jnp.dot(agg, w0_ref[...], preferred_element_type=jnp.float32) + b0_ref[...]
    h = jnp.maximum(h, 0.0)
    y2 = jnp.dot(h, w1_ref[...], preferred_element_type=jnp.float32) + b1_ref[...]
    t = (jnp.dot(y2, a_ref[...], preferred_element_type=jnp.float32)
         + jnp.dot(x_ref[...], bm_ref[...], preferred_element_type=jnp.float32)
         + jnp.dot(wf_ref[...], c_ref[...], preferred_element_type=jnp.float32)
         + mb0_ref[...])
    t = jnp.maximum(t, 0.0)
    o_ref[...] = jnp.dot(t, wo_ref[...], preferred_element_type=jnp.float32) + mbo_ref[...]


def _final(p, x, wf, w0, b0, w1, b1, a, bm, cm, mb0, wo, mbo):
    grid = (_N // _BR,)
    mat = pl.BlockSpec((_D, _D), lambda i: (0, 0))
    vec = pl.BlockSpec((1, _D), lambda i: (0, 0))
    row = pl.BlockSpec((_BR, _D), lambda i: (i, 0))
    return pl.pallas_call(
        _final_body,
        grid=grid,
        in_specs=[
            pl.BlockSpec((_NC, _BR, _D), lambda i: (0, i, 0)),
            row, row, mat, vec, mat, vec, mat, mat, mat, vec, mat, vec,
        ],
        out_specs=row,
        out_shape=jax.ShapeDtypeStruct((_N, _D), jnp.float32),
    )(p, x, wf, w0, b0, w1, b1, a, bm, cm, mb0, wo, mbo)


def kernel(node_attributes, edge_index, X, W,
           g0W0, g0b0, g0W1, g0b1, g1W0, g1b0, g1W1, g1b1,
           mW0, mb0, mWo, mbo):
    dpw = _EPAD // _NW
    dummy_src = (jnp.arange(_NW * dpw, dtype=jnp.int32) % _N).reshape(_NW, dpw)
    dummy_dst = jnp.broadcast_to(_N + jnp.arange(dpw, dtype=jnp.int32),
                                 (_NW, dpw))
    src3 = jnp.concatenate(
        [edge_index[0].reshape(_NW, _E // _NW), dummy_src], axis=1
    ).reshape(_NW, _NCH, _CH)
    dst3 = jnp.concatenate(
        [edge_index[1].reshape(_NW, _E // _NW), dummy_dst], axis=1
    ).reshape(_NW, _NCH, _CH)

    p1 = _segsum(node_attributes, src3, dst3)[:, :_N]
    y1 = _mlp(p1, g0W0, g0b0.reshape(1, _D), g0W1, g0b1.reshape(1, _D))
    p2 = _segsum(y1, src3, dst3)[:, :_N]

    a = mW0[:_D]
    bm = mW0[_D:2 * _D]
    cm = jnp.zeros((_D, _D), jnp.float32).at[:7].set(mW0[2 * _D:])
    wf = jnp.pad(W, ((0, 0), (0, _D - 7)))
    wo = jnp.zeros((_D, _D), jnp.float32).at[:, :4].set(mWo)
    mbo_p = jnp.zeros((1, _D), jnp.float32).at[0, :4].set(mbo)
    out = _final(p2, X, wf, g1W0, g1b0.reshape(1, _D), g1W1, g1b1.reshape(1, _D),
                 a, bm, cm, mb0.reshape(1, _D), wo, mbo_p)
    return out[:, :4]


# speedup vs baseline: 4.1133x; 1.3729x over previous
import jax
import jax.numpy as jnp
from jax import lax
from jax.experimental import pallas as pl
from jax.experimental.pallas import tpu as pltpu
from jax.experimental.pallas import tpu_sc as plsc

_N = 10000
_E = 320000
_D = 128
_NC = 2
_NS = 16
_NW = _NC * _NS
_CH = 128
_NCH = 80
_PC = 16
_NPH = _NCH // _PC
_EPW = _NCH * _CH
_EPAD = _NW * _EPW - _E
_NP = 10240
_RPT = _NP // _NS


def _segsum_body(y_hbm, src_hbm, dst_hbm, out_hbm,
                 src_v, dst_v, rows0, rows1, acc_sh,
                 gsem0, gsem1, ssem0, ssem1):
    c = lax.axis_index("c")
    s = lax.axis_index("s")
    w = s * _NC + c
    zrow = jnp.zeros((16,), jnp.float32)

    def _zfill(r, carry):
        for j in range(_D // 16):
            rows0[r, pl.ds(j * 16, 16)] = zrow
        return carry

    lax.fori_loop(0, _CH, _zfill, 0)
    base = s * _RPT

    def _zero(i, carry):
        pltpu.sync_copy(rows0, acc_sh.at[pl.ds(base + i * _CH, _CH)])
        return carry

    lax.fori_loop(0, _RPT // _CH, _zero, 0)
    plsc.subcore_barrier()

    def _pair(t, carry):
        i = t * 2

        @pl.when(i > 0)
        def _():
            pltpu.make_async_copy(rows1, acc_sh.at[dst_v.at[i - 1]], ssem1).wait()

        pltpu.async_copy(y_hbm.at[src_v.at[i + 1]], rows1, gsem1)
        pltpu.make_async_copy(y_hbm.at[src_v.at[i]], rows0, gsem0).wait()
        pltpu.async_copy(rows0, acc_sh.at[dst_v.at[i]], ssem0, add=True)
        pltpu.make_async_copy(rows0, acc_sh.at[dst_v.at[i]], ssem0).wait()

        @pl.when(i + 2 < _PC)
        def _():
            pltpu.async_copy(y_hbm.at[src_v.at[i + 2]], rows0, gsem0)

        pltpu.make_async_copy(y_hbm.at[src_v.at[i + 1]], rows1, gsem1).wait()
        pltpu.async_copy(rows1, acc_sh.at[dst_v.at[i + 1]], ssem1, add=True)
        return carry

    def _phase(p, carry):
        pltpu.sync_copy(src_hbm.at[w, pl.ds(p * _PC, _PC)], src_v)
        pltpu.sync_copy(dst_hbm.at[w, pl.ds(p * _PC, _PC)], dst_v)
        pltpu.async_copy(y_hbm.at[src_v.at[0]], rows0, gsem0)
        lax.fori_loop(0, _PC // 2, _pair, 0)
        pltpu.make_async_copy(rows1, acc_sh.at[dst_v.at[_PC - 1]], ssem1).wait()
        return carry

    lax.fori_loop(0, _NPH, _phase, 0)
    plsc.subcore_barrier()
    pltpu.sync_copy(acc_sh.at[pl.ds(base, _RPT)], out_hbm.at[c, pl.ds(base, _RPT)])


def _segsum(y, src3, dst3):
    mesh = plsc.VectorSubcoreMesh(core_axis_name="c", subcore_axis_name="s")
    f = pl.kernel(
        _segsum_body,
        mesh=mesh,
        out_type=jax.ShapeDtypeStruct((_NC, _NP, _D), jnp.float32),
        scratch_types=[
            pltpu.VMEM((_PC, _CH), jnp.int32),
            pltpu.VMEM((_PC, _CH), jnp.int32),
            pltpu.VMEM((_CH, _D), jnp.float32),
            pltpu.VMEM((_CH, _D), jnp.float32),
            pltpu.VMEM_SHARED((_NP, _D), jnp.float32),
            pltpu.SemaphoreType.DMA,
            pltpu.SemaphoreType.DMA,
            pltpu.SemaphoreType.DMA,
            pltpu.SemaphoreType.DMA,
        ],
    )
    return f(y, src3, dst3)


_BR = 2000


def _mlp_body(p_ref, w0_ref, b0_ref, w1_ref, b1_ref, o_ref):
    agg = p_ref[0] + p_ref[1]
    h = jnp.dot(agg, w0_ref[...], preferred_element_type=jnp.float32) + b0_ref[...]
    h = jnp.maximum(h, 0.0)
    o_ref[...] = jnp.dot(h, w1_ref[...], preferred_element_type=jnp.float32) + b1_ref[...]


def _mlp(p, w0, b0, w1, b1):
    grid = (_N // _BR,)
    return pl.pallas_call(
        _mlp_body,
        grid=grid,
        in_specs=[
            pl.BlockSpec((_NC, _BR, _D), lambda i: (0, i, 0)),
            pl.BlockSpec((_D, _D), lambda i: (0, 0)),
            pl.BlockSpec((1, _D), lambda i: (0, 0)),
            pl.BlockSpec((_D, _D), lambda i: (0, 0)),
            pl.BlockSpec((1, _D), lambda i: (0, 0)),
        ],
        out_specs=pl.BlockSpec((_BR, _D), lambda i: (i, 0)),
        out_shape=jax.ShapeDtypeStruct((_N, _D), jnp.float32),
    )(p, w0, b0, w1, b1)


def _final_body(p_ref, x_ref, wf_ref, w0_ref, b0_ref, w1_ref, b1_ref,
                a_ref, bm_ref, c_ref, mb0_ref, wo_ref, mbo_ref, o_ref):
    agg = p_ref[0] + p_ref[1]
    h = jnp.dot(agg, w0_ref[...], preferred_element_type=jnp.float32) + b0_ref[...]
    h = jnp.maximum(h, 0.0)
    y2 = jnp.dot(h, w1_ref[...], preferred_element_type=jnp.float32) + b1_ref[...]
    t = (jnp.dot(y2, a_ref[...], preferred_element_type=jnp.float32)
         + jnp.dot(x_ref[...], bm_ref[...], preferred_element_type=jnp.float32)
         + jnp.dot(wf_ref[...], c_ref[...], preferred_element_type=jnp.float32)
         + mb0_ref[...])
    t = jnp.maximum(t, 0.0)
    o_ref[...] = jnp.dot(t, wo_ref[...], preferred_element_type=jnp.float32) + mbo_ref[...]


def _final(p, x, wf, w0, b0, w1, b1, a, bm, cm, mb0, wo, mbo):
    grid = (_N // _BR,)
    mat = pl.BlockSpec((_D, _D), lambda i: (0, 0))
    vec = pl.BlockSpec((1, _D), lambda i: (0, 0))
    row = pl.BlockSpec((_BR, _D), lambda i: (i, 0))
    return pl.pallas_call(
        _final_body,
        grid=grid,
        in_specs=[
            pl.BlockSpec((_NC, _BR, _D), lambda i: (0, i, 0)),
            row, row, mat, vec, mat, vec, mat, mat, mat, vec, mat, vec,
        ],
        out_specs=row,
        out_shape=jax.ShapeDtypeStruct((_N, _D), jnp.float32),
    )(p, x, wf, w0, b0, w1, b1, a, bm, cm, mb0, wo, mbo)


def kernel(node_attributes, edge_index, X, W,
           g0W0, g0b0, g0W1, g0b1, g1W0, g1b0, g1W1, g1b1,
           mW0, mb0, mWo, mbo):
    dpw = _EPAD // _NW
    dummy_src = (jnp.arange(_NW * dpw, dtype=jnp.int32) % _N).reshape(_NW, dpw)
    dummy_dst = jnp.broadcast_to(_N + jnp.arange(dpw, dtype=jnp.int32),
                                 (_NW, dpw))
    src3 = jnp.concatenate(
        [edge_index[0].reshape(_NW, _E // _NW), dummy_src], axis=1
    ).reshape(_NW, _NCH, _CH)
    dst3 = jnp.concatenate(
        [edge_index[1].reshape(_NW, _E // _NW), dummy_dst], axis=1
    ).reshape(_NW, _NCH, _CH)

    p1 = _segsum(node_attributes, src3, dst3)[:, :_N]
    y1 = _mlp(p1, g0W0, g0b0.reshape(1, _D), g0W1, g0b1.reshape(1, _D))
    p2 = _segsum(y1, src3, dst3)[:, :_N]

    a = mW0[:_D]
    bm = mW0[_D:2 * _D]
    cm = jnp.zeros((_D, _D), jnp.float32).at[:7].set(mW0[2 * _D:])
    wf = jnp.pad(W, ((0, 0), (0, _D - 7)))
    wo = jnp.zeros((_D, _D), jnp.float32).at[:, :4].set(mWo)
    mbo_p = jnp.zeros((1, _D), jnp.float32).at[0, :4].set(mbo)
    out = _final(p2, X, wf, g1W0, g1b0.reshape(1, _D), g1W1, g1b1.reshape(1, _D),
                 a, bm, cm, mb0.reshape(1, _D), wo, mbo_p)
    return out[:, :4]


# speedup vs baseline: 4.4247x; 1.0757x over previous
import jax
import jax.numpy as jnp
from jax import lax
from jax.experimental import pallas as pl
from jax.experimental.pallas import tpu as pltpu
from jax.experimental.pallas import tpu_sc as plsc

_N = 10000
_E = 320000
_D = 128
_NC = 2
_NS = 16
_NW = _NC * _NS
_CH = 128
_NCH = 80
_PC = 40
_NPH = _NCH // _PC
_EPW = _NCH * _CH
_EPAD = _NW * _EPW - _E
_NP = 10240
_RPT = _NP // _NS


def _segsum_body(y_hbm, src_hbm, dst_hbm, out_hbm,
                 src_v, dst_v, rows0, rows1, acc_sh,
                 gsem0, gsem1, ssem0, ssem1):
    c = lax.axis_index("c")
    s = lax.axis_index("s")
    w = s * _NC + c
    zrow = jnp.zeros((16,), jnp.float32)

    def _zfill(r, carry):
        for j in range(_D // 16):
            rows0[r, pl.ds(j * 16, 16)] = zrow
        return carry

    lax.fori_loop(0, _CH, _zfill, 0)
    base = s * _RPT

    def _zero(i, carry):
        pltpu.sync_copy(rows0, acc_sh.at[pl.ds(base + i * _CH, _CH)])
        return carry

    lax.fori_loop(0, _RPT // _CH, _zero, 0)
    plsc.subcore_barrier()

    def _pair(t, carry):
        i = t * 2

        @pl.when(i > 0)
        def _():
            pltpu.make_async_copy(rows1, acc_sh.at[dst_v.at[i - 1]], ssem1).wait()

        pltpu.async_copy(y_hbm.at[src_v.at[i + 1]], rows1, gsem1)
        pltpu.make_async_copy(y_hbm.at[src_v.at[i]], rows0, gsem0).wait()
        pltpu.async_copy(rows0, acc_sh.at[dst_v.at[i]], ssem0, add=True)
        pltpu.make_async_copy(rows0, acc_sh.at[dst_v.at[i]], ssem0).wait()

        @pl.when(i + 2 < _PC)
        def _():
            pltpu.async_copy(y_hbm.at[src_v.at[i + 2]], rows0, gsem0)

        pltpu.make_async_copy(y_hbm.at[src_v.at[i + 1]], rows1, gsem1).wait()
        pltpu.async_copy(rows1, acc_sh.at[dst_v.at[i + 1]], ssem1, add=True)
        return carry

    def _phase(p, carry):
        pltpu.sync_copy(src_hbm.at[w, pl.ds(p * _PC, _PC)], src_v)
        pltpu.sync_copy(dst_hbm.at[w, pl.ds(p * _PC, _PC)], dst_v)
        pltpu.async_copy(y_hbm.at[src_v.at[0]], rows0, gsem0)
        lax.fori_loop(0, _PC // 2, _pair, 0)
        pltpu.make_async_copy(rows1, acc_sh.at[dst_v.at[_PC - 1]], ssem1).wait()
        return carry

    lax.fori_loop(0, _NPH, _phase, 0)
    plsc.subcore_barrier()
    pltpu.sync_copy(acc_sh.at[pl.ds(base, _RPT)], out_hbm.at[c, pl.ds(base, _RPT)])


def _segsum(y, src3, dst3):
    mesh = plsc.VectorSubcoreMesh(core_axis_name="c", subcore_axis_name="s")
    f = pl.kernel(
        _segsum_body,
        mesh=mesh,
        out_type=jax.ShapeDtypeStruct((_NC, _NP, _D), jnp.float32),
        scratch_types=[
            pltpu.VMEM((_PC, _CH), jnp.int32),
            pltpu.VMEM((_PC, _CH), jnp.int32),
            pltpu.VMEM((_CH, _D), jnp.float32),
            pltpu.VMEM((_CH, _D), jnp.float32),
            pltpu.VMEM_SHARED((_NP, _D), jnp.float32),
            pltpu.SemaphoreType.DMA,
            pltpu.SemaphoreType.DMA,
            pltpu.SemaphoreType.DMA,
            pltpu.SemaphoreType.DMA,
        ],
    )
    return f(y, src3, dst3)


_BR = 1280
_BRF = 2000


def _mlp_body(p_ref, w0_ref, b0_ref, w1_ref, b1_ref, o_ref):
    agg = p_ref[0] + p_ref[1]
    h = jnp.dot(agg, w0_ref[...], preferred_element_type=jnp.float32) + b0_ref[...]
    h = jnp.maximum(h, 0.0)
    o_ref[...] = jnp.dot(h, w1_ref[...], preferred_element_type=jnp.float32) + b1_ref[...]


def _mlp(p, w0, b0, w1, b1):
    grid = (_NP // _BR,)
    return pl.pallas_call(
        _mlp_body,
        grid=grid,
        in_specs=[
            pl.BlockSpec((_NC, _BR, _D), lambda i: (0, i, 0)),
            pl.BlockSpec((_D, _D), lambda i: (0, 0)),
            pl.BlockSpec((1, _D), lambda i: (0, 0)),
            pl.BlockSpec((_D, _D), lambda i: (0, 0)),
            pl.BlockSpec((1, _D), lambda i: (0, 0)),
        ],
        out_specs=pl.BlockSpec((_BR, _D), lambda i: (i, 0)),
        out_shape=jax.ShapeDtypeStruct((_NP, _D), jnp.float32),
    )(p, w0, b0, w1, b1)


def _final_body(p_ref, x_ref, wf_ref, w0_ref, b0_ref, w1_ref, b1_ref,
                a_ref, bm_ref, c_ref, mb0_ref, wo_ref, mbo_ref, o_ref):
    agg = p_ref[0] + p_ref[1]
    h = jnp.dot(agg, w0_ref[...], preferred_element_type=jnp.float32) + b0_ref[...]
    h = jnp.maximum(h, 0.0)
    y2 = jnp.dot(h, w1_ref[...], preferred_element_type=jnp.float32) + b1_ref[...]
    t = (jnp.dot(y2, a_ref[...], preferred_element_type=jnp.float32)
         + jnp.dot(x_ref[...], bm_ref[...], preferred_element_type=jnp.float32)
         + jnp.dot(wf_ref[...], c_ref[...], preferred_element_type=jnp.float32)
         + mb0_ref[...])
    t = jnp.maximum(t, 0.0)
    o_ref[...] = jnp.dot(t, wo_ref[...], preferred_element_type=jnp.float32) + mbo_ref[...]


def _final(p, x, wf, w0, b0, w1, b1, a, bm, cm, mb0, wo, mbo):
    grid = (_N // _BRF,)
    mat = pl.BlockSpec((_D, _D), lambda i: (0, 0))
    vec = pl.BlockSpec((1, _D), lambda i: (0, 0))
    row = pl.BlockSpec((_BRF, _D), lambda i: (i, 0))
    return pl.pallas_call(
        _final_body,
        grid=grid,
        in_specs=[
            pl.BlockSpec((_NC, _BRF, _D), lambda i: (0, i, 0)),
            row, row, mat, vec, mat, vec, mat, mat, mat, vec, mat, vec,
        ],
        out_specs=row,
        out_shape=jax.ShapeDtypeStruct((_N, _D), jnp.float32),
    )(p, x, wf, w0, b0, w1, b1, a, bm, cm, mb0, wo, mbo)


def kernel(node_attributes, edge_index, X, W,
           g0W0, g0b0, g0W1, g0b1, g1W0, g1b0, g1W1, g1b1,
           mW0, mb0, mWo, mbo):
    dpw = _EPAD // _NW
    dummy_src = (jnp.arange(_NW * dpw, dtype=jnp.int32) % _N).reshape(_NW, dpw)
    dummy_dst = jnp.broadcast_to(_N + jnp.arange(dpw, dtype=jnp.int32),
                                 (_NW, dpw))
    src3 = jnp.concatenate(
        [edge_index[0].reshape(_NW, _E // _NW), dummy_src], axis=1
    ).reshape(_NW, _NCH, _CH)
    dst3 = jnp.concatenate(
        [edge_index[1].reshape(_NW, _E // _NW), dummy_dst], axis=1
    ).reshape(_NW, _NCH, _CH)

    p1 = _segsum(node_attributes, src3, dst3)
    y1 = _mlp(p1, g0W0, g0b0.reshape(1, _D), g0W1, g0b1.reshape(1, _D))
    p2 = _segsum(y1, src3, dst3)[:, :_N]

    a = mW0[:_D]
    bm = mW0[_D:2 * _D]
    cm = jnp.zeros((_D, _D), jnp.float32).at[:7].set(mW0[2 * _D:])
    wf = jnp.pad(W, ((0, 0), (0, _D - 7)))
    wo = jnp.zeros((_D, _D), jnp.float32).at[:, :4].set(mWo)
    mbo_p = jnp.zeros((1, _D), jnp.float32).at[0, :4].set(mbo)
    out = _final(p2, X, wf, g1W0, g1b0.reshape(1, _D), g1W1, g1b1.reshape(1, _D),
                 a, bm, cm, mb0.reshape(1, _D), wo, mbo_p)
    return out[:, :4]


# speedup vs baseline: 4.5198x; 1.0215x over previous
import jax
import jax.numpy as jnp
from jax import lax
from jax.experimental import pallas as pl
from jax.experimental.pallas import tpu as pltpu
from jax.experimental.pallas import tpu_sc as plsc

_N = 10000
_E = 320000
_D = 128
_NC = 2
_NS = 16
_NW = _NC * _NS
_CH = 128
_NCH = 80
_PC = 40
_NPH = _NCH // _PC
_EPW = _NCH * _CH
_EPAD = _NW * _EPW - _E
_NP = 10240
_RPT = _NP // _NS


def _segsum_body(y_hbm, src_hbm, dst_hbm, out_hbm,
                 src_v, dst_v, rows0, rows1, acc_sh,
                 gsem0, gsem1, ssem0, ssem1):
    c = lax.axis_index("c")
    s = lax.axis_index("s")
    w = s * _NC + c
    zrow = jnp.zeros((16,), jnp.float32)

    def _zfill(r, carry):
        for j in range(_D // 16):
            rows0[r, pl.ds(j * 16, 16)] = zrow
        return carry

    lax.fori_loop(0, _CH, _zfill, 0)
    base = s * _RPT

    def _zero(i, carry):
        pltpu.sync_copy(rows0, acc_sh.at[pl.ds(base + i * _CH, _CH)])
        return carry

    lax.fori_loop(0, _RPT // _CH, _zero, 0)
    plsc.subcore_barrier()

    def _pair(t, carry):
        i = t * 2

        @pl.when(i > 0)
        def _():
            pltpu.make_async_copy(rows1, acc_sh.at[dst_v.at[i - 1]], ssem1).wait()

        pltpu.async_copy(y_hbm.at[src_v.at[i + 1]], rows1, gsem1)
        pltpu.make_async_copy(y_hbm.at[src_v.at[i]], rows0, gsem0).wait()
        pltpu.async_copy(rows0, acc_sh.at[dst_v.at[i]], ssem0, add=True)
        pltpu.make_async_copy(rows0, acc_sh.at[dst_v.at[i]], ssem0).wait()

        @pl.when(i + 2 < _PC)
        def _():
            pltpu.async_copy(y_hbm.at[src_v.at[i + 2]], rows0, gsem0)

        pltpu.make_async_copy(y_hbm.at[src_v.at[i + 1]], rows1, gsem1).wait()
        pltpu.async_copy(rows1, acc_sh.at[dst_v.at[i + 1]], ssem1, add=True)
        return carry

    def _phase(p, carry):
        pltpu.sync_copy(src_hbm.at[w, pl.ds(p * _PC, _PC)], src_v)
        pltpu.sync_copy(dst_hbm.at[w, pl.ds(p * _PC, _PC)], dst_v)
        pltpu.async_copy(y_hbm.at[src_v.at[0]], rows0, gsem0)
        lax.fori_loop(0, _PC // 2, _pair, 0)
        pltpu.make_async_copy(rows1, acc_sh.at[dst_v.at[_PC - 1]], ssem1).wait()
        return carry

    lax.fori_loop(0, _NPH, _phase, 0)
    plsc.subcore_barrier()
    pltpu.sync_copy(acc_sh.at[pl.ds(base, _RPT)], out_hbm.at[c, pl.ds(base, _RPT)])


def _segsum(y, src3, dst3):
    mesh = plsc.VectorSubcoreMesh(core_axis_name="c", subcore_axis_name="s")
    f = pl.kernel(
        _segsum_body,
        mesh=mesh,
        out_type=jax.ShapeDtypeStruct((_NC, _NP, _D), jnp.float32),
        scratch_types=[
            pltpu.VMEM((_PC, _CH), jnp.int32),
            pltpu.VMEM((_PC, _CH), jnp.int32),
            pltpu.VMEM((_CH, _D), jnp.float32),
            pltpu.VMEM((_CH, _D), jnp.float32),
            pltpu.VMEM_SHARED((_NP, _D), jnp.float32),
            pltpu.SemaphoreType.DMA,
            pltpu.SemaphoreType.DMA,
            pltpu.SemaphoreType.DMA,
            pltpu.SemaphoreType.DMA,
        ],
    )
    return f(y, src3, dst3)


_BR = 1280
_BRF = 2000


def _mlp_body(p_ref, w0_ref, b0_ref, w1_ref, b1_ref, o_ref):
    agg = p_ref[0] + p_ref[1]
    h = jnp.dot(agg, w0_ref[...], preferred_element_type=jnp.float32) + b0_ref[...]
    h = jnp.maximum(h, 0.0)
    o_ref[...] = jnp.dot(h, w1_ref[...], preferred_element_type=jnp.float32) + b1_ref[...]


def _mlp(p, w0, b0, w1, b1):
    grid = (_NP // _BR,)
    return pl.pallas_call(
        _mlp_body,
        grid=grid,
        in_specs=[
            pl.BlockSpec((_NC, _BR, _D), lambda i: (0, i, 0)),
            pl.BlockSpec((_D, _D), lambda i: (0, 0)),
            pl.BlockSpec((1, _D), lambda i: (0, 0)),
            pl.BlockSpec((_D, _D), lambda i: (0, 0)),
            pl.BlockSpec((1, _D), lambda i: (0, 0)),
        ],
        out_specs=pl.BlockSpec((_BR, _D), lambda i: (i, 0)),
        out_shape=jax.ShapeDtypeStruct((_NP, _D), jnp.float32),
    )(p, w0, b0, w1, b1)


def _final_body(p_ref, x_ref, wf_ref, w0_ref, b0_ref, w1_ref, b1_ref,
                a_ref, bm_ref, c_ref, mb0_ref, wo_ref, mbo_ref, o_ref):
    agg = p_ref[0] + p_ref[1]
    h = jnp.dot(agg, w0_ref[...], preferred_element_type=jnp.float32) + b0_ref[...]
    h = jnp.maximum(h, 0.0)
    y2 = jnp.dot(h, w1_ref[...], preferred_element_type=jnp.float32) + b1_ref[...]
    t = (jnp.dot(y2, a_ref[...], preferred_element_type=jnp.float32)
         + jnp.dot(x_ref[...], bm_ref[...], preferred_element_type=jnp.float32)
         + jnp.dot(wf_ref[...], c_ref[...], preferred_element_type=jnp.float32)
         + mb0_ref[...])
    t = jnp.maximum(t, 0.0)
    o_ref[...] = jnp.dot(t, wo_ref[...], preferred_element_type=jnp.float32) + mbo_ref[...]


def _final(p, x, wf, w0, b0, w1, b1, a, bm, cm, mb0, wo, mbo):
    grid = (_N // _BRF,)
    mat = pl.BlockSpec((_D, _D), lambda i: (0, 0))
    vec = pl.BlockSpec((1, _D), lambda i: (0, 0))
    row = pl.BlockSpec((_BRF, _D), lambda i: (i, 0))
    return pl.pallas_call(
        _final_body,
        grid=grid,
        in_specs=[
            pl.BlockSpec((_NC, _BRF, _D), lambda i: (0, i, 0)),
            row, row, mat, vec, mat, vec, mat, mat, mat, vec, mat, vec,
        ],
        out_specs=row,
        out_shape=jax.ShapeDtypeStruct((_N, _D), jnp.float32),
    )(p, x, wf, w0, b0, w1, b1, a, bm, cm, mb0, wo, mbo)


def kernel(node_attributes, edge_index, X, W,
           g0W0, g0b0, g0W1, g0b1, g1W0, g1b0, g1W1, g1b1,
           mW0, mb0, mWo, mbo):
    dpw = _EPAD // _NW
    dummy_src = (jnp.arange(_NW * dpw, dtype=jnp.int32) % _N).reshape(_NW, dpw)
    dummy_dst = jnp.broadcast_to(_N + jnp.arange(dpw, dtype=jnp.int32),
                                 (_NW, dpw))
    src3 = jnp.concatenate(
        [edge_index[0].reshape(_NW, _E // _NW), dummy_src], axis=1
    ).reshape(_NW, _NCH, _CH)
    dst3 = jnp.concatenate(
        [edge_index[1].reshape(_NW, _E // _NW), dummy_dst], axis=1
    ).reshape(_NW, _NCH, _CH)

    p1 = _segsum(node_attributes, src3, dst3)
    y1 = _mlp(p1, g0W0, g0b0.reshape(1, _D), g0W1, g0b1.reshape(1, _D))
    p2 = _segsum(y1, src3, dst3)

    a = mW0[:_D]
    bm = mW0[_D:2 * _D]
    cm = jnp.zeros((_D, _D), jnp.float32).at[:7].set(mW0[2 * _D:])
    wf = jnp.pad(W, ((0, 0), (0, _D - 7)))
    wo = jnp.zeros((_D, _D), jnp.float32).at[:, :4].set(mWo)
    mbo_p = jnp.zeros((1, _D), jnp.float32).at[0, :4].set(mbo)
    out = _final(p2, X, wf, g1W0, g1b0.reshape(1, _D), g1W1, g1b1.reshape(1, _D),
                 a, bm, cm, mb0.reshape(1, _D), wo, mbo_p)
    return out[:, :4]
